# baseline (speedup 1.0000x reference)
import functools

import jax
import jax.numpy as jnp
from jax import lax
from jax.experimental import pallas as pl
from jax.experimental.pallas import tpu as pltpu
from jax.experimental.pallas import tpu_sc as plsc

N1 = 1000000
DIM = 16
B = 98304

PACK = 128 // DIM
N_LINES = N1 // PACK

NUM_CORES = 2
NUM_SUBCORES = 16
NUM_WORKERS = NUM_CORES * NUM_SUBCORES
B_PER_W = B // NUM_WORKERS
CHUNK = 128
N_CHUNKS = B_PER_W // CHUNK
B_ROWS = B // 128
W_ROWS = B_ROWS // NUM_WORKERS


def _sc_gather_dot(tab_s, tab_t, idx_s, idx_t):
    mesh = plsc.VectorSubcoreMesh(core_axis_name="c", subcore_axis_name="s")

    @functools.partial(
        pl.kernel,
        mesh=mesh,
        compiler_params=pltpu.CompilerParams(needs_layout_passes=False),
        out_type=jax.ShapeDtypeStruct((B_ROWS, 128), jnp.float32),
        scratch_types=[
            pltpu.VMEM((W_ROWS, 128), jnp.int32),
            pltpu.VMEM((W_ROWS, 128), jnp.int32),
            pltpu.VMEM((W_ROWS, 128), jnp.int32),
            pltpu.VMEM((W_ROWS, 128), jnp.int32),
            pltpu.VMEM((CHUNK, 128), jnp.float32),
            pltpu.VMEM((CHUNK, 128), jnp.float32),
            pltpu.VMEM((W_ROWS, 128), jnp.float32),
            pltpu.SemaphoreType.DMA,
            pltpu.SemaphoreType.DMA,
        ],
    )
    def gather_dot(src_tab, tgt_tab, src_idx, tgt_idx, out_ip,
                   idx_s_v, idx_t_v, line_s_v, line_t_v,
                   rows_s_v, rows_t_v, ip_v, sem_s, sem_t):
        wid = lax.axis_index("s") * NUM_CORES + lax.axis_index("c")
        base = wid * W_ROWS
        pltpu.sync_copy(src_idx.at[pl.ds(base, W_ROWS)], idx_s_v)
        pltpu.sync_copy(tgt_idx.at[pl.ds(base, W_ROWS)], idx_t_v)

        iota16 = lax.broadcasted_iota(jnp.int32, (16,), 0)

        @pl.loop(0, N_CHUNKS)
        def _(j):
            for g in range(8):
                sl = pl.ds(g * 16, 16)
                line_s_v[j, sl] = lax.shift_right_logical(idx_s_v[j, sl], 3)
                line_t_v[j, sl] = lax.shift_right_logical(idx_t_v[j, sl], 3)
            cp_s = pltpu.async_copy(src_tab.at[line_s_v.at[j]], rows_s_v, sem_s)
            cp_t = pltpu.async_copy(tgt_tab.at[line_t_v.at[j]], rows_t_v, sem_t)
            cp_s.wait()
            cp_t.wait()
            for g in range(8):
                sl = pl.ds(g * 16, 16)
                rows16 = g * 16 + iota16
                lane_s = (idx_s_v[j, sl] & 7) * 16
                lane_t = (idx_t_v[j, sl] & 7) * 16
                acc = jnp.zeros((16,), jnp.float32)
                for d in range(DIM):
                    a = plsc.load_gather(rows_s_v, [rows16, lane_s + d])
                    b = plsc.load_gather(rows_t_v, [rows16, lane_t + d])
                    acc = acc + a * b
                ip_v[j, sl] = acc

        pltpu.sync_copy(ip_v, out_ip.at[pl.ds(base, W_ROWS)])

    return gather_dot(tab_s, tab_t, idx_s, idx_t)


def _tc_loss_body(ip_ref, lab_ref, out_ref):
    z = lab_ref[...] * ip_ref[...]
    out_ref[...] = jnp.sum(jax.nn.log_sigmoid(z)).reshape(1, 1)


def _tc_loss_sum(ip, lab):
    return pl.pallas_call(
        _tc_loss_body,
        out_shape=jax.ShapeDtypeStruct((1, 1), jnp.float32),
    )(ip, lab)


def kernel(source_node, target_node, label, nodes_embed, context_nodes_embed):
    tab_s = jnp.reshape(nodes_embed, (N_LINES, 128))
    tab_t = jnp.reshape(context_nodes_embed, (N_LINES, 128))
    idx_s = jnp.reshape(source_node.astype(jnp.int32), (B_ROWS, 128))
    idx_t = jnp.reshape(target_node.astype(jnp.int32), (B_ROWS, 128))
    ip = _sc_gather_dot(tab_s, tab_t, idx_s, idx_t)
    total = _tc_loss_sum(ip, jnp.reshape(label, (B_ROWS, 128)))
    return -total[0, 0] / jnp.float32(B)


# speedup vs baseline: 1.1924x; 1.1924x over previous
import functools

import jax
import jax.numpy as jnp
from jax import lax
from jax.experimental import pallas as pl
from jax.experimental.pallas import tpu as pltpu
from jax.experimental.pallas import tpu_sc as plsc

N1 = 1000000
DIM = 16
B = 98304

PACK = 128 // DIM
N_LINES = N1 // PACK

NUM_CORES = 2
NUM_SUBCORES = 16
NUM_WORKERS = NUM_CORES * NUM_SUBCORES
B_PER_W = B // NUM_WORKERS
CHUNK = 128
N_CHUNKS = B_PER_W // CHUNK
B_ROWS = B // 128
W_ROWS = B_ROWS // NUM_WORKERS


def _sc_gather_dot(tab_s, tab_t, idx_s, idx_t):
    mesh = plsc.VectorSubcoreMesh(core_axis_name="c", subcore_axis_name="s")

    @functools.partial(
        pl.kernel,
        mesh=mesh,
        compiler_params=pltpu.CompilerParams(needs_layout_passes=False),
        out_type=jax.ShapeDtypeStruct((B_ROWS, 128), jnp.float32),
        scratch_types=[
            pltpu.VMEM((W_ROWS, 128), jnp.int32),
            pltpu.VMEM((W_ROWS, 128), jnp.int32),
            pltpu.VMEM((W_ROWS, 128), jnp.int32),
            pltpu.VMEM((W_ROWS, 128), jnp.int32),
            pltpu.VMEM((CHUNK, 128), jnp.float32),
            pltpu.VMEM((CHUNK, 128), jnp.float32),
            pltpu.VMEM((W_ROWS, 128), jnp.float32),
            pltpu.SemaphoreType.DMA,
            pltpu.SemaphoreType.DMA,
        ],
    )
    def gather_dot(src_tab, tgt_tab, src_idx, tgt_idx, out_ip,
                   idx_s_v, idx_t_v, line_s_v, line_t_v,
                   rows_s_v, rows_t_v, ip_v, sem_s, sem_t):
        wid = lax.axis_index("s") * NUM_CORES + lax.axis_index("c")
        base = wid * W_ROWS
        pltpu.sync_copy(src_idx.at[pl.ds(base, W_ROWS)], idx_s_v)
        pltpu.sync_copy(tgt_idx.at[pl.ds(base, W_ROWS)], idx_t_v)

        iota16 = lax.broadcasted_iota(jnp.int32, (16,), 0)

        @pl.loop(0, N_CHUNKS)
        def _(j):
            for g in range(8):
                sl = pl.ds(g * 16, 16)
                line_s_v[j, sl] = lax.shift_right_logical(idx_s_v[j, sl], 3)
                line_t_v[j, sl] = lax.shift_right_logical(idx_t_v[j, sl], 3)
            cp_s = pltpu.async_copy(src_tab.at[line_s_v.at[j]], rows_s_v, sem_s)
            cp_t = pltpu.async_copy(tgt_tab.at[line_t_v.at[j]], rows_t_v, sem_t)
            cp_s.wait()
            cp_t.wait()
            for g in range(8):
                sl = pl.ds(g * 16, 16)
                rows16 = g * 16 + iota16
                lane_s = (idx_s_v[j, sl] & 7) * 16
                lane_t = (idx_t_v[j, sl] & 7) * 16
                acc = jnp.zeros((16,), jnp.float32)
                for d in range(DIM):
                    a = plsc.load_gather(rows_s_v, [rows16, lane_s + d])
                    b = plsc.load_gather(rows_t_v, [rows16, lane_t + d])
                    acc = acc + a * b
                ip_v[j, sl] = acc

        pltpu.sync_copy(ip_v, out_ip.at[pl.ds(base, W_ROWS)])

    return gather_dot(tab_s, tab_t, idx_s, idx_t)


_R_LANES = 2048
_R_LINES = _R_LANES // PACK
_R_STEPS = (N_LINES + _R_LINES - 1) // _R_LINES


def _relayout_body(s_ref, t_ref, os_ref, ot_ref):
    def repack(x):
        t = jnp.transpose(x).reshape(_R_LINES, PACK, DIM)
        return jnp.concatenate([t[:, j, :] for j in range(PACK)], axis=1)

    os_ref[...] = repack(s_ref[...])
    ot_ref[...] = repack(t_ref[...])


def _relayout_pair(tab_s_t, tab_t_t):
    return pl.pallas_call(
        _relayout_body,
        grid=(_R_STEPS,),
        in_specs=[
            pl.BlockSpec((DIM, _R_LANES), lambda k: (0, k)),
            pl.BlockSpec((DIM, _R_LANES), lambda k: (0, k)),
        ],
        out_specs=[
            pl.BlockSpec((_R_LINES, 128), lambda k: (k, 0)),
            pl.BlockSpec((_R_LINES, 128), lambda k: (k, 0)),
        ],
        out_shape=[
            jax.ShapeDtypeStruct((N_LINES, 128), jnp.float32),
            jax.ShapeDtypeStruct((N_LINES, 128), jnp.float32),
        ],
    )(tab_s_t, tab_t_t)


def _tc_loss_body(ip_ref, lab_ref, out_ref):
    z = lab_ref[...] * ip_ref[...]
    out_ref[...] = jnp.sum(jax.nn.log_sigmoid(z)).reshape(1, 1)


def _tc_loss_sum(ip, lab):
    return pl.pallas_call(
        _tc_loss_body,
        out_shape=jax.ShapeDtypeStruct((1, 1), jnp.float32),
    )(ip, lab)


def kernel(source_node, target_node, label, nodes_embed, context_nodes_embed):
    tab_s, tab_t = _relayout_pair(jnp.transpose(nodes_embed),
                                  jnp.transpose(context_nodes_embed))
    idx_s = jnp.reshape(source_node.astype(jnp.int32), (B_ROWS, 128))
    idx_t = jnp.reshape(target_node.astype(jnp.int32), (B_ROWS, 128))
    ip = _sc_gather_dot(tab_s, tab_t, idx_s, idx_t)
    total = _tc_loss_sum(ip, jnp.reshape(label, (B_ROWS, 128)))
    return -total[0, 0] / jnp.float32(B)


# speedup vs baseline: 1.2352x; 1.0359x over previous
import functools

import jax
import jax.numpy as jnp
from jax import lax
from jax.experimental import pallas as pl
from jax.experimental.pallas import tpu as pltpu
from jax.experimental.pallas import tpu_sc as plsc

N1 = 1000000
DIM = 16
B = 98304

PACK = 128 // DIM
N_LINES = N1 // PACK

NUM_CORES = 2
NUM_SUBCORES = 16
NUM_WORKERS = NUM_CORES * NUM_SUBCORES
B_PER_W = B // NUM_WORKERS
CHUNK = 128
N_CHUNKS = B_PER_W // CHUNK
B_ROWS = B // 128
W_ROWS = B_ROWS // NUM_WORKERS


def _sc_gather_dot(tab_s, tab_t, idx_s, idx_t):
    mesh = plsc.VectorSubcoreMesh(core_axis_name="c", subcore_axis_name="s")

    @functools.partial(
        pl.kernel,
        mesh=mesh,
        compiler_params=pltpu.CompilerParams(needs_layout_passes=False),
        out_type=jax.ShapeDtypeStruct((B_ROWS, 128), jnp.float32),
        scratch_types=[
            pltpu.VMEM((W_ROWS, 128), jnp.int32),
            pltpu.VMEM((W_ROWS, 128), jnp.int32),
            pltpu.VMEM((W_ROWS, 128), jnp.int32),
            pltpu.VMEM((W_ROWS, 128), jnp.int32),
            pltpu.VMEM((CHUNK, 128), jnp.float32),
            pltpu.VMEM((CHUNK, 128), jnp.float32),
            pltpu.VMEM((W_ROWS, 128), jnp.float32),
            pltpu.SemaphoreType.DMA,
            pltpu.SemaphoreType.DMA,
        ],
    )
    def gather_dot(src_tab, tgt_tab, src_idx, tgt_idx, out_ip,
                   idx_s_v, idx_t_v, line_s_v, line_t_v,
                   rows_s_v, rows_t_v, ip_v, sem_s, sem_t):
        wid = lax.axis_index("s") * NUM_CORES + lax.axis_index("c")
        base = wid * W_ROWS
        pltpu.sync_copy(src_idx.at[pl.ds(base, W_ROWS)], idx_s_v)
        pltpu.sync_copy(tgt_idx.at[pl.ds(base, W_ROWS)], idx_t_v)

        iota16 = lax.broadcasted_iota(jnp.int32, (16,), 0)

        @pl.loop(0, N_CHUNKS)
        def _(j):
            for g in range(8):
                sl = pl.ds(g * 16, 16)
                line_s_v[j, sl] = lax.shift_right_logical(idx_s_v[j, sl], 3)
                line_t_v[j, sl] = lax.shift_right_logical(idx_t_v[j, sl], 3)
            cp_s = pltpu.async_copy(src_tab.at[line_s_v.at[j]], rows_s_v, sem_s)
            cp_t = pltpu.async_copy(tgt_tab.at[line_t_v.at[j]], rows_t_v, sem_t)
            cp_s.wait()
            cp_t.wait()
            for g in range(8):
                sl = pl.ds(g * 16, 16)
                rows16 = g * 16 + iota16
                lane_s = (idx_s_v[j, sl] & 7) * 16
                lane_t = (idx_t_v[j, sl] & 7) * 16
                acc = jnp.zeros((16,), jnp.float32)
                for d in range(DIM):
                    a = plsc.load_gather(rows_s_v, [rows16, lane_s + d])
                    b = plsc.load_gather(rows_t_v, [rows16, lane_t + d])
                    acc = acc + a * b
                ip_v[j, sl] = acc

        pltpu.sync_copy(ip_v, out_ip.at[pl.ds(base, W_ROWS)])

    return gather_dot(tab_s, tab_t, idx_s, idx_t)


_R_LANES = 2048
_R_LINES = _R_LANES // PACK
_R_STEPS = (N_LINES + _R_LINES - 1) // _R_LINES


def _relayout_body(s_ref, t_ref, os_ref, ot_ref):
    lane = jax.numpy.arange(128)
    placed = [jnp.where((lane[None, :] == j * DIM + jnp.arange(DIM)[:, None]),
                        jnp.float32(1), jnp.float32(0)) for j in range(PACK)]

    def repack(x):
        t = jnp.transpose(x).reshape(_R_LINES, PACK, DIM)
        acc = jnp.zeros((_R_LINES, 128), jnp.float32)
        for j in range(PACK):
            acc = acc + lax.dot_general(
                t[:, j, :], placed[j], (((1,), (0,)), ((), ())),
                preferred_element_type=jnp.float32)
        return acc

    os_ref[...] = repack(s_ref[...])
    ot_ref[...] = repack(t_ref[...])


def _relayout_pair(tab_s_t, tab_t_t):
    return pl.pallas_call(
        _relayout_body,
        grid=(_R_STEPS,),
        in_specs=[
            pl.BlockSpec((DIM, _R_LANES), lambda k: (0, k)),
            pl.BlockSpec((DIM, _R_LANES), lambda k: (0, k)),
        ],
        out_specs=[
            pl.BlockSpec((_R_LINES, 128), lambda k: (k, 0)),
            pl.BlockSpec((_R_LINES, 128), lambda k: (k, 0)),
        ],
        out_shape=[
            jax.ShapeDtypeStruct((N_LINES, 128), jnp.float32),
            jax.ShapeDtypeStruct((N_LINES, 128), jnp.float32),
        ],
    )(tab_s_t, tab_t_t)


def _tc_loss_body(ip_ref, lab_ref, out_ref):
    z = lab_ref[...] * ip_ref[...]
    out_ref[...] = jnp.sum(jax.nn.log_sigmoid(z)).reshape(1, 1)


def _tc_loss_sum(ip, lab):
    return pl.pallas_call(
        _tc_loss_body,
        out_shape=jax.ShapeDtypeStruct((1, 1), jnp.float32),
    )(ip, lab)


def kernel(source_node, target_node, label, nodes_embed, context_nodes_embed):
    tab_s, tab_t = _relayout_pair(jnp.transpose(nodes_embed),
                                  jnp.transpose(context_nodes_embed))
    idx_s = jnp.reshape(source_node.astype(jnp.int32), (B_ROWS, 128))
    idx_t = jnp.reshape(target_node.astype(jnp.int32), (B_ROWS, 128))
    ip = _sc_gather_dot(tab_s, tab_t, idx_s, idx_t)
    total = _tc_loss_sum(ip, jnp.reshape(label, (B_ROWS, 128)))
    return -total[0, 0] / jnp.float32(B)
